# baseline (speedup 1.0000x reference)
import functools

import jax
import jax.numpy as jnp
from jax import lax
from jax.experimental import pallas as pl
from jax.experimental.pallas import tpu as pltpu
from jax.experimental.pallas import tpu_sc as plsc

_B = 4096
_QL = 20
_DL = 200
_EMBED = 64
_LANES = 128
_NC = 2
_NS = 16
_NW = _NC * _NS
_RPW = _B // _NW
_DCH = 2
_QCH = 8


def _sum_rows(buf_v, par_v, base, n, outb_v, out_row):
    def tok(j, accs):
        off = par_v[pl.ds(base + j, 16)][0] * 64
        return tuple(
            accs[c] + buf_v[base + j, pl.ds(off + 16 * c, 16)]
            for c in range(4)
        )
    accs = lax.fori_loop(
        0, n, tok, tuple(jnp.zeros((16,), jnp.float32) for _ in range(4))
    )
    for c in range(4):
        outb_v[out_row, pl.ds(16 * c, 16)] = accs[c]


def _pool_body(qh_ref, qp_ref, dh_ref, dp_ref, qt_ref, dt_ref,
               qo_ref, do_ref,
               qidx_v, qpar_v, qbuf_v, didx_v, dpar_v, dbuf_v, outb_v, sem):
    wid = lax.axis_index("s") * _NC + lax.axis_index("c")

    def run_phase(half_hbm, par_hbm, tab_hbm, out_hbm, seq_len, ch_rows,
                  idx_v, par_v, buf_v):
        k = ch_rows * seq_len
        nch = _RPW // ch_rows
        base = wid * _RPW * seq_len

        def chunk(ch, carry):
            pltpu.sync_copy(half_hbm.at[pl.ds(base + ch * k, k)], idx_v)
            pltpu.sync_copy(par_hbm.at[pl.ds(base + ch * k, k)],
                            par_v.at[pl.ds(0, k)])
            pltpu.async_copy(tab_hbm.at[idx_v], buf_v, sem).wait()
            for r in range(ch_rows):
                _sum_rows(buf_v, par_v, r * seq_len, seq_len, outb_v,
                          ch * ch_rows + r)
            return carry

        lax.fori_loop(0, nch, chunk, 0)
        pltpu.sync_copy(outb_v, out_hbm.at[pl.ds(wid * _RPW, _RPW)])

    run_phase(qh_ref, qp_ref, qt_ref, qo_ref, _QL, _QCH,
              qidx_v, qpar_v, qbuf_v)
    run_phase(dh_ref, dp_ref, dt_ref, do_ref, _DL, _DCH,
              didx_v, dpar_v, dbuf_v)


def _sc_pool(qh, qp, dh, dp, q_view, d_view):
    mesh = plsc.VectorSubcoreMesh(core_axis_name="c", subcore_axis_name="s")
    out_type = (
        jax.ShapeDtypeStruct((_B, _LANES), jnp.float32),
        jax.ShapeDtypeStruct((_B, _LANES), jnp.float32),
    )
    scratch = [
        pltpu.VMEM((_QCH * _QL,), jnp.int32),
        pltpu.VMEM((_QCH * _QL + 16,), jnp.int32),
        pltpu.VMEM((_QCH * _QL, _LANES), jnp.float32),
        pltpu.VMEM((_DCH * _DL,), jnp.int32),
        pltpu.VMEM((_DCH * _DL + 16,), jnp.int32),
        pltpu.VMEM((_DCH * _DL, _LANES), jnp.float32),
        pltpu.VMEM((_RPW, _LANES), jnp.float32),
        pltpu.SemaphoreType.DMA,
    ]
    f = pl.kernel(_pool_body, out_type=out_type, mesh=mesh,
                  scratch_types=scratch)
    return f(qh, qp, dh, dp, q_view, d_view)


def _epilogue_body(qs_ref, ds_ref, o_ref):
    q = jnp.tanh(qs_ref[:, :_EMBED] * (1.0 / _QL))
    d = jnp.tanh(ds_ref[:, :_EMBED] * (1.0 / _DL))
    qn = jnp.sqrt(jnp.sum(q * q, axis=1, keepdims=True))
    dn = jnp.sqrt(jnp.sum(d * d, axis=1, keepdims=True))
    q = q / jnp.maximum(qn, 1e-12)
    d = d / jnp.maximum(dn, 1e-12)
    o_ref[...] = jnp.sum(q * d, axis=1)


def _tc_epilogue(q_sum, d_sum):
    return pl.pallas_call(
        _epilogue_body,
        out_shape=jax.ShapeDtypeStruct((_B,), jnp.float32),
    )(q_sum, d_sum)


def kernel(qs, ds, rels, q_table, d_table):
    del rels
    q_view = q_table.reshape(-1, _LANES)
    d_view = d_table.reshape(-1, _LANES)
    qf = qs.reshape(-1)
    df = ds.reshape(-1)
    q_sum, d_sum = _sc_pool(
        qf >> 1, qf & 1, df >> 1, df & 1, q_view, d_view
    )
    return _tc_epilogue(q_sum, d_sum)


# speedup vs baseline: 1.3184x; 1.3184x over previous
import functools

import jax
import jax.numpy as jnp
from jax import lax
from jax.experimental import pallas as pl
from jax.experimental.pallas import tpu as pltpu
from jax.experimental.pallas import tpu_sc as plsc

_B = 4096
_QL = 20
_DL = 200
_EMBED = 64
_OUTW = 128
_NC = 2
_NS = 16
_NW = _NC * _NS
_RPW = _B // _NW


def _sum_rows(buf_v, base, n, outb_v, out_row):
    def tok(j, accs):
        return tuple(
            accs[c] + buf_v[base + j, pl.ds(16 * c, 16)] for c in range(4)
        )
    accs = lax.fori_loop(
        0, n, tok, tuple(jnp.zeros((16,), jnp.float32) for _ in range(4))
    )
    for c in range(4):
        outb_v[out_row, pl.ds(16 * c, 16)] = accs[c]


def _make_phase_body(seq_len, ch_rows):
    k = ch_rows * seq_len
    nch = _RPW // ch_rows

    def body(idx_ref, tab_ref, out_ref, idx_v, buf_v, outb_v, sem):
        wid = lax.axis_index("s") * _NC + lax.axis_index("c")
        base = wid * _RPW * seq_len

        def chunk(ch, carry):
            pltpu.sync_copy(idx_ref.at[pl.ds(base + ch * k, k)], idx_v)
            pltpu.async_copy(tab_ref.at[idx_v], buf_v, sem).wait()
            for r in range(ch_rows):
                _sum_rows(buf_v, r * seq_len, seq_len, outb_v,
                          ch * ch_rows + r)
            return carry

        lax.fori_loop(0, nch, chunk, 0)
        pltpu.sync_copy(outb_v, out_ref.at[pl.ds(wid * _RPW, _RPW)])

    return body


def _sc_phase(idx_flat, table, seq_len, ch_rows):
    mesh = plsc.VectorSubcoreMesh(core_axis_name="c", subcore_axis_name="s")
    k = ch_rows * seq_len
    scratch = [
        pltpu.VMEM((k,), jnp.int32),
        pltpu.VMEM((k, _EMBED), jnp.float32),
        pltpu.VMEM((_RPW, _OUTW), jnp.float32),
        pltpu.SemaphoreType.DMA,
    ]
    f = pl.kernel(_make_phase_body(seq_len, ch_rows),
                  out_type=jax.ShapeDtypeStruct((_B, _OUTW), jnp.float32),
                  mesh=mesh, scratch_types=scratch,
                  compiler_params=pltpu.CompilerParams(
                      use_tc_tiling_on_sc=False))
    return f(idx_flat, table)


def _epilogue_body(qs_ref, ds_ref, o_ref):
    q = jnp.tanh(qs_ref[:, :_EMBED] * (1.0 / _QL))
    d = jnp.tanh(ds_ref[:, :_EMBED] * (1.0 / _DL))
    qn = jnp.sqrt(jnp.sum(q * q, axis=1, keepdims=True))
    dn = jnp.sqrt(jnp.sum(d * d, axis=1, keepdims=True))
    q = q / jnp.maximum(qn, 1e-12)
    d = d / jnp.maximum(dn, 1e-12)
    o_ref[...] = jnp.sum(q * d, axis=1)


def _tc_epilogue(q_sum, d_sum):
    return pl.pallas_call(
        _epilogue_body,
        out_shape=jax.ShapeDtypeStruct((_B,), jnp.float32),
    )(q_sum, d_sum)


def kernel(qs, ds, rels, q_table, d_table):
    del rels
    q_sum = _sc_phase(qs.reshape(-1), q_table, _QL, 16)
    d_sum = _sc_phase(ds.reshape(-1), d_table, _DL, 4)
    return _tc_epilogue(q_sum, d_sum)
